# baseline (speedup 1.0000x reference)
import functools

import jax
import jax.numpy as jnp
from jax import lax
from jax.experimental import pallas as pl
from jax.experimental.pallas import tpu as pltpu
from jax.experimental.pallas import tpu_sc as plsc

_DIM = 128
_BASE = 10000.0
_NC = 2
_NS = 16
_NW = _NC * _NS
_ROWS = 32
_WPP = _ROWS * _DIM
_CHUNK = 4


def _rope_tables(sq):
    freqs = 1.0 / (_BASE ** (jnp.arange(0, _DIM, 2, dtype=jnp.float32) / _DIM))
    t = jnp.arange(sq, dtype=jnp.float32)
    f = jnp.outer(t, freqs)
    cos = jnp.cos(f)
    sin = jnp.sin(f)
    c_tab = jnp.repeat(cos, 2, axis=1)
    s_tab = jnp.stack([-sin, sin], axis=-1).reshape(sq, _DIM)
    return c_tab.reshape(-1), s_tab.reshape(-1)


@jax.jit
def _sc_rope(query, key):
    sq = query.shape[0]
    n_pos_w = sq // _NW
    n_chunks = n_pos_w // _CHUNK
    cw = _CHUNK * _WPP
    tw = n_pos_w * _DIM

    c_tab, s_tab = _rope_tables(sq)
    qf = query.reshape(-1)
    kf = key.reshape(-1)

    mesh = plsc.VectorSubcoreMesh(core_axis_name="c", subcore_axis_name="s")

    @functools.partial(
        pl.kernel,
        mesh=mesh,
        out_type=[
            jax.ShapeDtypeStruct(qf.shape, qf.dtype),
            jax.ShapeDtypeStruct(kf.shape, kf.dtype),
        ],
        scratch_types=[
            pltpu.VMEM((cw,), jnp.float32),
            pltpu.VMEM((cw,), jnp.float32),
            pltpu.VMEM((tw,), jnp.float32),
            pltpu.VMEM((tw,), jnp.float32),
        ],
        compiler_params=pltpu.CompilerParams(needs_layout_passes=False),
    )
    def k(q_hbm, k_hbm, c_hbm, s_hbm, qo_hbm, ko_hbm, inb, outb, ctab, stab):
        swap = lax.iota(jnp.int32, 16) ^ 1
        wid = lax.axis_index("s") * _NC + lax.axis_index("c")
        wbase = wid * n_pos_w * _WPP
        tbase = wid * tw
        pltpu.sync_copy(c_hbm.at[pl.ds(tbase, tw)], ctab)
        pltpu.sync_copy(s_hbm.at[pl.ds(tbase, tw)], stab)

        def do_chunk(src, dst, ci):
            base = wbase + ci * cw
            pltpu.sync_copy(src.at[pl.ds(base, cw)], inb)

            def body(i, carry):
                db = i * _DIM
                cb = ci * _CHUNK * _DIM + (i >> 5) * _DIM
                for v in range(8):
                    x = inb[pl.ds(db + 16 * v, 16)]
                    c = ctab[pl.ds(cb + 16 * v, 16)]
                    s = stab[pl.ds(cb + 16 * v, 16)]
                    idx = jnp.full((16,), db + 16 * v, jnp.int32) + swap
                    xsw = plsc.load_gather(inb, [idx])
                    outb[pl.ds(db + 16 * v, 16)] = x * c + xsw * s
                return carry

            lax.fori_loop(0, _CHUNK * _ROWS, body, 0)
            pltpu.sync_copy(outb, dst.at[pl.ds(base, cw)])

        def chunks(src, dst):
            lax.fori_loop(0, n_chunks,
                          lambda ci, c: (do_chunk(src, dst, ci), c)[1], 0)

        chunks(q_hbm, qo_hbm)
        chunks(k_hbm, ko_hbm)

    qo, ko = k(qf, kf, c_tab, s_tab)
    return qo.reshape(query.shape), ko.reshape(key.shape)


def kernel(query, key):
    return _sc_rope(query, key)


# speedup vs baseline: 8.9602x; 8.9602x over previous
import functools

import jax
import jax.numpy as jnp
from jax.experimental import pallas as pl

_DIM = 128
_BASE = 10000.0


def _rope_tables(sq):
    freqs = 1.0 / (_BASE ** (jnp.arange(0, _DIM, 2, dtype=jnp.float32) / _DIM))
    t = jnp.arange(sq, dtype=jnp.float32)
    f = jnp.outer(t, freqs)
    cos = jnp.cos(f)
    sin = jnp.sin(f)
    c_tab = jnp.repeat(cos, 2, axis=1)
    s_tab = jnp.stack([-sin, sin], axis=-1).reshape(sq, _DIM)
    return c_tab, s_tab


def _rope_kernel(q_ref, k_ref, c_ref, s_ref, qo_ref, ko_ref):
    c = c_ref[...][:, None, :]
    s = s_ref[...][:, None, :]
    lane = jax.lax.broadcasted_iota(jnp.int32, (1, 1, _DIM), 2)
    even = (lane % 2) == 0

    def apply(x):
        xsw = jnp.where(even, jnp.roll(x, -1, axis=-1), jnp.roll(x, 1, axis=-1))
        return x * c + xsw * s

    qo_ref[...] = apply(q_ref[...])
    ko_ref[...] = apply(k_ref[...])


@functools.partial(jax.jit, static_argnames=("interpret",))
def _rope(query, key, interpret=False):
    sq, b, nh, hh = query.shape
    rows = b * nh
    q2 = query.reshape(sq, rows, hh)
    k2 = key.reshape(sq, rows, hh)
    c_tab, s_tab = _rope_tables(sq)

    s_blk = 256
    grid = (sq // s_blk,)
    xspec = pl.BlockSpec((s_blk, rows, hh), lambda i: (i, 0, 0))
    tspec = pl.BlockSpec((s_blk, hh), lambda i: (i, 0))

    qo, ko = pl.pallas_call(
        _rope_kernel,
        grid=grid,
        in_specs=[xspec, xspec, tspec, tspec],
        out_specs=[xspec, xspec],
        out_shape=[
            jax.ShapeDtypeStruct((sq, rows, hh), query.dtype),
            jax.ShapeDtypeStruct((sq, rows, hh), key.dtype),
        ],
        interpret=interpret,
    )(q2, k2, c_tab, s_tab)
    return qo.reshape(sq, b, nh, hh), ko.reshape(sq, b, nh, hh)


def kernel(query, key):
    return _rope(query, key)
